# Initial kernel scaffold; baseline (speedup 1.0000x reference)
import jax
import jax.numpy as jnp
from jax.experimental import pallas as pl


def kernel(pcd_xyz, rgb_xyz, rgb_features, params):
    raise NotImplementedError("write your pallas kernel here")


# speedup vs baseline: 1.0026x; 1.0026x over previous
import functools

import jax
import jax.numpy as jnp
from jax.experimental import pallas as pl

_B, _N, _M = 2, 8192, 2048


def _mlp_body(x_ref, *rest, n_layers, ns, ts):
    out_ref = rest[-1]
    ws = rest[:-1]
    y = x_ref[0]
    for i in range(n_layers):
        w = ws[2 * i][...]
        b = ws[2 * i + 1][...]
        y = jnp.dot(y, w, preferred_element_type=jnp.float32,
                    precision=jax.lax.Precision.HIGHEST)
        y = jnp.maximum(y + b, 0.0)
    if ns > 1:
        y = y.reshape(ts, ns, y.shape[-1]).max(axis=1)
    out_ref[0] = y


def _mlp_max(x_flat, Wts, bs, ns, ts):
    b_, p, cin = x_flat.shape
    s = p // ns
    cout = Wts[-1].shape[1]
    n_layers = len(Wts)
    grid = (b_, s // ts)
    in_specs = [pl.BlockSpec((1, ts * ns, cin), lambda bi, pi: (bi, pi, 0))]
    args = []
    for w, bias in zip(Wts, bs):
        args.append(w)
        args.append(bias.reshape(1, -1))
        in_specs.append(pl.BlockSpec(w.shape, lambda bi, pi: (0, 0)))
        in_specs.append(pl.BlockSpec((1, bias.shape[0]), lambda bi, pi: (0, 0)))
    out_spec = pl.BlockSpec((1, ts, cout), lambda bi, pi: (bi, pi, 0))
    fn = pl.pallas_call(
        functools.partial(_mlp_body, n_layers=n_layers, ns=ns, ts=ts),
        grid=grid,
        in_specs=in_specs,
        out_specs=out_spec,
        out_shape=jax.ShapeDtypeStruct((b_, s, cout), jnp.float32),
    )
    return fn(x_flat, *args)


def _fps(xyz, npoint):
    b, n, _ = xyz.shape
    def body(i, state):
        dists, inds, far = state
        inds = inds.at[:, i].set(far)
        cen = jnp.take_along_axis(xyz, far[:, None, None], axis=1)
        d = jnp.sum((xyz - cen) ** 2, axis=-1)
        dists = jnp.minimum(dists, d)
        far = jnp.argmax(dists, axis=-1).astype(jnp.int32)
        return (dists, inds, far)
    state = (jnp.full((b, n), 1e10, jnp.float32),
             jnp.zeros((b, npoint), jnp.int32), jnp.zeros((b,), jnp.int32))
    _, inds, _ = jax.lax.fori_loop(0, npoint, body, state)
    return inds


def _gather_pts(x, idx):
    return jnp.take_along_axis(x, idx[..., None], axis=1)


def _ball_query(radius, nsample, xyz, new_xyz):
    n = xyz.shape[1]
    d2 = jnp.sum((new_xyz[:, :, None, :] - xyz[:, None, :, :]) ** 2, axis=-1)
    key = jnp.where(d2 <= radius * radius,
                    jnp.arange(n, dtype=jnp.int32)[None, None, :], jnp.int32(n))
    vals, _ = jax.lax.top_k(-key, nsample)
    idx = -vals
    idx = jnp.where(idx == n, idx[..., :1], idx)
    idx = jnp.where(idx == n, 0, idx)
    return idx


def _sa_module(xyz, features, npoint, radius, nsample, p):
    Ws, bs = p
    inds = _fps(jax.lax.stop_gradient(xyz), npoint)
    new_xyz = _gather_pts(xyz, inds)
    idx = _ball_query(radius, nsample, xyz, new_xyz)
    b, s, ns = idx.shape
    flat = idx.reshape(b, s * ns)
    gx = _gather_pts(xyz, flat).reshape(b, s, ns, 3) - new_xyz[:, :, None, :]
    gx = gx / radius
    if features is not None:
        xf = jnp.transpose(features, (0, 2, 1))
        gf = _gather_pts(xf, flat).reshape(b, s, ns, features.shape[1])
        g = jnp.concatenate([gx, gf], axis=-1)
    else:
        g = gx
    g = g.reshape(b, s * ns, g.shape[-1])
    Wts = [w.T for w in Ws]
    f = _mlp_max(g, Wts, bs, ns=ns, ts=128)
    return new_xyz, jnp.transpose(f, (0, 2, 1)), inds.astype(jnp.int64)


def _fp_module(unknown, known, unknown_feats, known_feats, p):
    Ws, bs = p
    b, n, _ = unknown.shape
    d2 = jnp.sum((jax.lax.stop_gradient(unknown)[:, :, None, :]
                  - jax.lax.stop_gradient(known)[:, None, :, :]) ** 2, axis=-1)
    _, idx = jax.lax.top_k(-d2, 3)
    idx = idx.astype(jnp.int32)
    flat = idx.reshape(b, n * 3)
    k3 = _gather_pts(known, flat).reshape(b, n, 3, 3)
    d = jnp.sum((unknown[:, :, None, :] - k3) ** 2, axis=-1)
    w = 1.0 / (d + 1e-8)
    w = w / jnp.sum(w, axis=-1, keepdims=True)
    kf = jnp.transpose(known_feats, (0, 2, 1))
    gf = _gather_pts(kf, flat).reshape(b, n, 3, known_feats.shape[1])
    interp = jnp.sum(gf * w[..., None], axis=2)
    if unknown_feats is not None:
        newf = jnp.concatenate([interp, jnp.transpose(unknown_feats, (0, 2, 1))],
                               axis=-1)
    else:
        newf = interp
    Wts = [w_.T for w_ in Ws]
    f = _mlp_max(newf, Wts, bs, ns=1, ts=2048)
    return jnp.transpose(f, (0, 2, 1))


def kernel(pcd_xyz, rgb_xyz, rgb_features, params):
    xyz = pcd_xyz[..., :3]
    sa1_xyz, sa1_f, sa1_inds = _sa_module(xyz, None, 4096, 0.1, 64, params['sa1'])
    seed = jnp.concatenate([sa1_f, jnp.zeros_like(sa1_f)], axis=1)
    rgbf = jnp.concatenate([jnp.zeros_like(rgb_features), rgb_features], axis=1)
    xyz2 = jnp.concatenate([sa1_xyz, rgb_xyz], axis=1)
    feat2 = jnp.concatenate([seed, rgbf], axis=2)
    sa2_xyz, sa2_f, sa2_inds = _sa_module(xyz2, feat2, 2048, 0.2, 64, params['sa2'])
    f = _fp_module(sa1_xyz, sa2_xyz, sa1_f, sa2_f, params['fp3'])
    f = _fp_module(pcd_xyz[..., :3], sa1_xyz, None, f, params['fp4'])
    return (sa1_inds, sa1_xyz, sa1_f, sa2_inds, sa2_xyz, sa2_f, f, pcd_xyz)


# speedup vs baseline: 1.6634x; 1.6591x over previous
import functools

import jax
import jax.numpy as jnp
from jax.experimental import pallas as pl
from jax.experimental.pallas import tpu as pltpu

_B, _N, _M = 2, 8192, 2048


def _mlp_body(x_ref, *rest, n_layers, ns, ts):
    out_ref = rest[-1]
    ws = rest[:-1]
    y = x_ref[0]
    for i in range(n_layers):
        w = ws[2 * i][...]
        b = ws[2 * i + 1][...]
        y = jnp.dot(y, w, preferred_element_type=jnp.float32,
                    precision=jax.lax.Precision.HIGHEST)
        y = jnp.maximum(y + b, 0.0)
    if ns > 1:
        y = y.reshape(ts, ns, y.shape[-1]).max(axis=1)
    out_ref[0] = y


def _mlp_max(x_flat, Wts, bs, ns, ts):
    b_, p, cin = x_flat.shape
    s = p // ns
    cout = Wts[-1].shape[1]
    n_layers = len(Wts)
    grid = (b_, s // ts)
    in_specs = [pl.BlockSpec((1, ts * ns, cin), lambda bi, pi: (bi, pi, 0))]
    args = []
    for w, bias in zip(Wts, bs):
        args.append(w)
        args.append(bias.reshape(1, -1))
        in_specs.append(pl.BlockSpec(w.shape, lambda bi, pi: (0, 0)))
        in_specs.append(pl.BlockSpec((1, bias.shape[0]), lambda bi, pi: (0, 0)))
    out_spec = pl.BlockSpec((1, ts, cout), lambda bi, pi: (bi, pi, 0))
    fn = pl.pallas_call(
        functools.partial(_mlp_body, n_layers=n_layers, ns=ns, ts=ts),
        grid=grid,
        in_specs=in_specs,
        out_specs=out_spec,
        out_shape=jax.ShapeDtypeStruct((b_, s, cout), jnp.float32),
    )
    return fn(x_flat, *args)


def _fps_body(xyz_ref, out_ref, cen_ref, *, npoint, n8):
    x = xyz_ref[0, 0]
    y = xyz_ref[0, 1]
    z = xyz_ref[0, 2]
    n = 8 * n8
    fidx = (jax.lax.broadcasted_iota(jnp.int32, (8, n8), 0) * n8
            + jax.lax.broadcasted_iota(jnp.int32, (8, n8), 1))

    def body(i, carry):
        dists, far = carry
        out_ref[0, 0, i] = far
        mask = fidx == far
        cx = jnp.sum(jnp.where(mask, x, 0.0))
        cy = jnp.sum(jnp.where(mask, y, 0.0))
        cz = jnp.sum(jnp.where(mask, z, 0.0))
        cen_ref[0, 0, i] = cx
        cen_ref[0, 1, i] = cy
        cen_ref[0, 2, i] = cz
        dx = x - cx
        dy = y - cy
        dz = z - cz
        d = (dx * dx + dy * dy) + dz * dz
        dists = jnp.minimum(dists, d)
        m = jnp.max(dists)
        far2 = jnp.min(jnp.where(dists == m, fidx, n))
        return dists, far2

    jax.lax.fori_loop(0, npoint, body,
                      (jnp.full((8, n8), 1e10, jnp.float32), jnp.int32(0)),
                      unroll=2)


def _fps(xyz, npoint):
    b, n, _ = xyz.shape
    n8 = n // 8
    xr = jnp.transpose(xyz, (0, 2, 1)).reshape(b, 3, 8, n8)
    fn = pl.pallas_call(
        functools.partial(_fps_body, npoint=npoint, n8=n8),
        grid=(b,),
        in_specs=[pl.BlockSpec((1, 3, 8, n8), lambda bi: (bi, 0, 0, 0))],
        out_specs=[pl.BlockSpec((1, 1, npoint), lambda bi: (bi, 0, 0),
                                memory_space=pltpu.SMEM),
                   pl.BlockSpec((1, 3, npoint), lambda bi: (bi, 0, 0),
                                memory_space=pltpu.SMEM)],
        out_shape=[jax.ShapeDtypeStruct((b, 1, npoint), jnp.int32),
                   jax.ShapeDtypeStruct((b, 3, npoint), jnp.float32)],
    )
    inds, cen = fn(xr)
    return inds[:, 0, :], jnp.transpose(cen, (0, 2, 1))


def _gather_pts(x, idx):
    return jnp.take_along_axis(x, idx[..., None], axis=1)


def _ball_query(radius, nsample, xyz, new_xyz):
    n = xyz.shape[1]
    d2 = jnp.sum((new_xyz[:, :, None, :] - xyz[:, None, :, :]) ** 2, axis=-1)
    key = jnp.where(d2 <= radius * radius,
                    jnp.arange(n, dtype=jnp.int32)[None, None, :], jnp.int32(n))
    vals, _ = jax.lax.top_k(-key, nsample)
    idx = -vals
    idx = jnp.where(idx == n, idx[..., :1], idx)
    idx = jnp.where(idx == n, 0, idx)
    return idx


def _sa_module(xyz, features, npoint, radius, nsample, p):
    Ws, bs = p
    inds, new_xyz = _fps(jax.lax.stop_gradient(xyz), npoint)
    idx = _ball_query(radius, nsample, xyz, new_xyz)
    b, s, ns = idx.shape
    flat = idx.reshape(b, s * ns)
    gx = _gather_pts(xyz, flat).reshape(b, s, ns, 3) - new_xyz[:, :, None, :]
    gx = gx / radius
    if features is not None:
        xf = jnp.transpose(features, (0, 2, 1))
        gf = _gather_pts(xf, flat).reshape(b, s, ns, features.shape[1])
        g = jnp.concatenate([gx, gf], axis=-1)
    else:
        g = gx
    g = g.reshape(b, s * ns, g.shape[-1])
    Wts = [w.T for w in Ws]
    f = _mlp_max(g, Wts, bs, ns=ns, ts=128)
    return new_xyz, jnp.transpose(f, (0, 2, 1)), inds.astype(jnp.int64)


def _fp_module(unknown, known, unknown_feats, known_feats, p):
    Ws, bs = p
    b, n, _ = unknown.shape
    d2 = jnp.sum((jax.lax.stop_gradient(unknown)[:, :, None, :]
                  - jax.lax.stop_gradient(known)[:, None, :, :]) ** 2, axis=-1)
    _, idx = jax.lax.top_k(-d2, 3)
    idx = idx.astype(jnp.int32)
    flat = idx.reshape(b, n * 3)
    k3 = _gather_pts(known, flat).reshape(b, n, 3, 3)
    d = jnp.sum((unknown[:, :, None, :] - k3) ** 2, axis=-1)
    w = 1.0 / (d + 1e-8)
    w = w / jnp.sum(w, axis=-1, keepdims=True)
    kf = jnp.transpose(known_feats, (0, 2, 1))
    gf = _gather_pts(kf, flat).reshape(b, n, 3, known_feats.shape[1])
    interp = jnp.sum(gf * w[..., None], axis=2)
    if unknown_feats is not None:
        newf = jnp.concatenate([interp, jnp.transpose(unknown_feats, (0, 2, 1))],
                               axis=-1)
    else:
        newf = interp
    Wts = [w_.T for w_ in Ws]
    f = _mlp_max(newf, Wts, bs, ns=1, ts=2048)
    return jnp.transpose(f, (0, 2, 1))


def kernel(pcd_xyz, rgb_xyz, rgb_features, params):
    xyz = pcd_xyz[..., :3]
    sa1_xyz, sa1_f, sa1_inds = _sa_module(xyz, None, 4096, 0.1, 64, params['sa1'])
    seed = jnp.concatenate([sa1_f, jnp.zeros_like(sa1_f)], axis=1)
    rgbf = jnp.concatenate([jnp.zeros_like(rgb_features), rgb_features], axis=1)
    xyz2 = jnp.concatenate([sa1_xyz, rgb_xyz], axis=1)
    feat2 = jnp.concatenate([seed, rgbf], axis=2)
    sa2_xyz, sa2_f, sa2_inds = _sa_module(xyz2, feat2, 2048, 0.2, 64, params['sa2'])
    f = _fp_module(sa1_xyz, sa2_xyz, sa1_f, sa2_f, params['fp3'])
    f = _fp_module(pcd_xyz[..., :3], sa1_xyz, None, f, params['fp4'])
    return (sa1_inds, sa1_xyz, sa1_f, sa2_inds, sa2_xyz, sa2_f, f, pcd_xyz)


# speedup vs baseline: 4.2363x; 2.5468x over previous
import functools

import jax
import jax.numpy as jnp
from jax import lax
from jax.experimental import pallas as pl
from jax.experimental.pallas import tpu as pltpu, tpu_sc as plsc

_NC, _NS, _L = 2, 16, 16
_NW = _NC * _NS


def _mlp_body(*refs, nx, n_layers, ns, ts):
    out_ref = refs[-1]
    xs = refs[:nx]
    ws = refs[nx:-1]
    y = None
    for i in range(nx):
        t = jnp.dot(xs[i][0], ws[i][...], preferred_element_type=jnp.float32,
                    precision=lax.Precision.HIGHEST)
        y = t if y is None else y + t
    y = jnp.maximum(y + ws[nx][...], 0.0)
    k = nx + 1
    for _ in range(n_layers - 1):
        y = jnp.dot(y, ws[k][...], preferred_element_type=jnp.float32,
                    precision=lax.Precision.HIGHEST)
        y = jnp.maximum(y + ws[k + 1][...], 0.0)
        k += 2
    if ns > 1:
        y = y.reshape(ts, ns, y.shape[-1]).max(axis=1)
    out_ref[0] = y


def _mlp_max(xs, W1s, b1, Wr, br, ns, ts):
    nx = len(xs)
    b_, p, _ = xs[0].shape
    s = p // ns
    n_layers = 1 + len(Wr)
    cout = (Wr[-1] if Wr else W1s[0]).shape[1]
    grid = (b_, s // ts)
    in_specs = []
    args = []
    for x in xs:
        in_specs.append(pl.BlockSpec((1, ts * ns, x.shape[2]),
                                     lambda bi, pi: (bi, pi, 0)))
        args.append(x)
    weights = list(W1s) + [b1.reshape(1, -1)]
    for w, bias in zip(Wr, br):
        weights.append(w)
        weights.append(bias.reshape(1, -1))
    for w in weights:
        in_specs.append(pl.BlockSpec(w.shape, lambda bi, pi: (0, 0)))
        args.append(w)
    fn = pl.pallas_call(
        functools.partial(_mlp_body, nx=nx, n_layers=n_layers, ns=ns, ts=ts),
        grid=grid,
        in_specs=in_specs,
        out_specs=pl.BlockSpec((1, ts, cout), lambda bi, pi: (bi, pi, 0)),
        out_shape=jax.ShapeDtypeStruct((b_, s, cout), jnp.float32),
    )
    return fn(*args)


def _fps_body(xyz_ref, out_ref, cen_ref, *, npoint, n8):
    x = xyz_ref[0, 0]
    y = xyz_ref[0, 1]
    z = xyz_ref[0, 2]
    n = 8 * n8
    fidx = (lax.broadcasted_iota(jnp.int32, (8, n8), 0) * n8
            + lax.broadcasted_iota(jnp.int32, (8, n8), 1))

    def body(i, carry):
        dists, far = carry
        out_ref[0, 0, i] = far
        mask = fidx == far
        cx = jnp.sum(jnp.where(mask, x, 0.0))
        cy = jnp.sum(jnp.where(mask, y, 0.0))
        cz = jnp.sum(jnp.where(mask, z, 0.0))
        cen_ref[0, 0, i] = cx
        cen_ref[0, 1, i] = cy
        cen_ref[0, 2, i] = cz
        dx = x - cx
        dy = y - cy
        dz = z - cz
        d = (dx * dx + dy * dy) + dz * dz
        dists = jnp.minimum(dists, d)
        m = jnp.max(dists)
        far2 = jnp.min(jnp.where(dists == m, fidx, n))
        return dists, far2

    lax.fori_loop(0, npoint, body,
                  (jnp.full((8, n8), 1e10, jnp.float32), jnp.int32(0)),
                  unroll=2)


def _fps(xyz, npoint):
    b, n, _ = xyz.shape
    n8 = n // 8
    xr = jnp.transpose(xyz, (0, 2, 1)).reshape(b, 3, 8, n8)
    fn = pl.pallas_call(
        functools.partial(_fps_body, npoint=npoint, n8=n8),
        grid=(b,),
        in_specs=[pl.BlockSpec((1, 3, 8, n8), lambda bi: (bi, 0, 0, 0))],
        out_specs=[pl.BlockSpec((1, 1, npoint), lambda bi: (bi, 0, 0),
                                memory_space=pltpu.SMEM),
                   pl.BlockSpec((1, 3, npoint), lambda bi: (bi, 0, 0),
                                memory_space=pltpu.SMEM)],
        out_shape=[jax.ShapeDtypeStruct((b, 1, npoint), jnp.int32),
                   jax.ShapeDtypeStruct((b, 3, npoint), jnp.float32)],
    )
    inds, cen3 = fn(xr)
    return inds[:, 0, :], cen3


def _ball_query_sc(xyz3, cen3, radius, ns):
    B, _, n = xyz3.shape
    S = cen3.shape[2]
    gpt = (S // _L) // _NW
    r2 = jnp.float32(radius * radius)
    mesh = plsc.VectorSubcoreMesh(core_axis_name="c", subcore_axis_name="s")

    @functools.partial(
        pl.kernel,
        out_type=[jax.ShapeDtypeStruct((B * S * ns,), jnp.int32),
                  jax.ShapeDtypeStruct((B * S * ns,), jnp.float32),
                  jax.ShapeDtypeStruct((B * S * ns,), jnp.float32),
                  jax.ShapeDtypeStruct((B * S * ns,), jnp.float32)],
        mesh=mesh,
        scratch_types=[pltpu.VMEM((3, n), jnp.float32),
                       pltpu.VMEM((3, S), jnp.float32),
                       pltpu.VMEM((_L * ns,), jnp.int32),
                       pltpu.VMEM((_L * ns,), jnp.float32),
                       pltpu.VMEM((_L * ns,), jnp.float32),
                       pltpu.VMEM((_L * ns,), jnp.float32)],
        compiler_params=pltpu.CompilerParams(needs_layout_passes=False),
    )
    def bq(xyz_hbm, cen_hbm, idx_out, gxx_out, gxy_out, gxz_out,
           xyz_v, cen_v, oi, ogx, ogy, ogz):
        wid = lax.axis_index("s") * _NC + lax.axis_index("c")
        lanes = lax.iota(jnp.int32, _L)
        lbase = lanes * ns
        for b in range(B):
            pltpu.sync_copy(xyz_hbm.at[b], xyz_v)
            pltpu.sync_copy(cen_hbm.at[b], cen_v)

            def tbody(t, _):
                s0 = (t * _NW + wid) * _L
                cx = cen_v[0, pl.ds(s0, _L)]
                cy = cen_v[1, pl.ds(s0, _L)]
                cz = cen_v[2, pl.ds(s0, _L)]

                def jbody(jc, cnt):
                    j0 = jc * _L
                    xv = xyz_v[0, pl.ds(j0, _L)]
                    yv = xyz_v[1, pl.ds(j0, _L)]
                    zv = xyz_v[2, pl.ds(j0, _L)]
                    for l in range(_L):
                        dx = xv[l] - cx
                        dy = yv[l] - cy
                        dz = zv[l] - cz
                        d2 = (dx * dx + dy * dy) + dz * dz
                        hit = (d2 <= r2) & (cnt < ns)
                        pos = lbase + jnp.minimum(cnt, ns - 1)
                        jv = jnp.full((_L,), j0 + l, jnp.int32)
                        plsc.store_scatter(oi, [pos], jv, mask=hit)
                        plsc.store_scatter(ogx, [pos], dx, mask=hit)
                        plsc.store_scatter(ogy, [pos], dy, mask=hit)
                        plsc.store_scatter(ogz, [pos], dz, mask=hit)
                        cnt = cnt + hit.astype(jnp.int32)
                    return cnt

                cnt = lax.fori_loop(0, n // _L, jbody,
                                    jnp.zeros((_L,), jnp.int32))

                p0 = xyz_v[0, pl.ds(0, _L)]
                p1 = xyz_v[1, pl.ds(0, _L)]
                p2 = xyz_v[2, pl.ds(0, _L)]
                has = cnt > 0
                fi = jnp.where(has, plsc.load_gather(oi, [lbase]), 0)
                fx = jnp.where(has, plsc.load_gather(ogx, [lbase]), p0[0] - cx)
                fy = jnp.where(has, plsc.load_gather(ogy, [lbase]), p1[0] - cy)
                fz = jnp.where(has, plsc.load_gather(ogz, [lbase]), p2[0] - cz)

                def pbody(p, _):
                    m = cnt <= p
                    pv = lbase + p
                    plsc.store_scatter(oi, [pv], fi, mask=m)
                    plsc.store_scatter(ogx, [pv], fx, mask=m)
                    plsc.store_scatter(ogy, [pv], fy, mask=m)
                    plsc.store_scatter(ogz, [pv], fz, mask=m)
                    return 0

                lax.fori_loop(0, ns, pbody, 0)

                off = (b * S + s0) * ns
                pltpu.sync_copy(oi, idx_out.at[pl.ds(off, _L * ns)])
                pltpu.sync_copy(ogx, gxx_out.at[pl.ds(off, _L * ns)])
                pltpu.sync_copy(ogy, gxy_out.at[pl.ds(off, _L * ns)])
                pltpu.sync_copy(ogz, gxz_out.at[pl.ds(off, _L * ns)])
                return 0

            lax.fori_loop(0, gpt, tbody, 0)

    return bq(xyz3, cen3)


def _gather_rows_sc(table, idxg):
    V, C = table.shape
    R = idxg.shape[0]
    rpt = R // _NW
    nch = rpt // 128
    mesh = plsc.VectorSubcoreMesh(core_axis_name="c", subcore_axis_name="s")

    @functools.partial(
        pl.kernel,
        out_type=jax.ShapeDtypeStruct((R, C), jnp.float32),
        mesh=mesh,
        scratch_types=[pltpu.VMEM((128,), jnp.int32),
                       pltpu.VMEM((128, C), jnp.float32),
                       pltpu.SemaphoreType.DMA],
        compiler_params=pltpu.CompilerParams(needs_layout_passes=False),
    )
    def gk(table_hbm, idx_hbm, out_hbm, idx_v, rows_v, sem):
        wid = lax.axis_index("s") * _NC + lax.axis_index("c")
        base = wid * rpt

        def cbody(c, _):
            r0 = base + c * 128
            pltpu.sync_copy(idx_hbm.at[pl.ds(r0, 128)], idx_v)
            pltpu.async_copy(table_hbm.at[idx_v], rows_v, sem).wait()
            pltpu.sync_copy(rows_v, out_hbm.at[pl.ds(r0, 128), :])
            return 0

        lax.fori_loop(0, nch, cbody, 0)

    return gk(table, idxg)


def _sa_module(xyz, feats_pm, npoint, radius, nsample, p):
    Ws, bs = p
    b, n, _ = xyz.shape
    inds, cen3 = _fps(lax.stop_gradient(xyz), npoint)
    xyz3 = jnp.transpose(xyz, (0, 2, 1))
    idx, gxx, gxy, gxz = _ball_query_sc(xyz3, cen3, radius, nsample)
    idx = idx.reshape(b, npoint, nsample)
    gxf = jnp.stack([gxx, gxy, gxz], axis=-1).reshape(b, npoint * nsample, 3)
    Wts = [w.T for w in Ws]
    W1 = Wts[0]
    if feats_pm is not None:
        cf = feats_pm.shape[2]
        idxg = (idx.reshape(b, npoint * nsample)
                + (jnp.arange(b, dtype=jnp.int32) * n)[:, None]).reshape(-1)
        gf = _gather_rows_sc(feats_pm.reshape(b * n, cf), idxg)
        gf = gf.reshape(b, npoint * nsample, cf)
        xs = [gxf, gf]
        W1s = [W1[:3] / radius, W1[3:]]
    else:
        xs = [gxf]
        W1s = [W1 / radius]
    f = _mlp_max(xs, W1s, bs[0], Wts[1:], bs[1:], ns=nsample, ts=128)
    new_xyz = jnp.transpose(cen3, (0, 2, 1))
    return new_xyz, jnp.transpose(f, (0, 2, 1)), inds.astype(jnp.int64)


def _fp_module(unknown, known, unknown_feats, known_feats, p):
    Ws, bs = p
    b, n, _ = unknown.shape
    d2 = jnp.sum((lax.stop_gradient(unknown)[:, :, None, :]
                  - lax.stop_gradient(known)[:, None, :, :]) ** 2, axis=-1)
    _, idx = lax.top_k(-d2, 3)
    idx = idx.astype(jnp.int32)
    flat = idx.reshape(b, n * 3)
    k3 = jnp.take_along_axis(known, flat[..., None], axis=1).reshape(b, n, 3, 3)
    d = jnp.sum((unknown[:, :, None, :] - k3) ** 2, axis=-1)
    w = 1.0 / (d + 1e-8)
    w = w / jnp.sum(w, axis=-1, keepdims=True)
    kf = jnp.transpose(known_feats, (0, 2, 1))
    gf = jnp.take_along_axis(kf, flat[..., None], axis=1).reshape(
        b, n, 3, known_feats.shape[1])
    interp = jnp.sum(gf * w[..., None], axis=2)
    Wts = [w_.T for w_ in Ws]
    if unknown_feats is not None:
        xs = [interp, jnp.transpose(unknown_feats, (0, 2, 1))]
        ci = interp.shape[2]
        W1s = [Wts[0][:ci], Wts[0][ci:]]
    else:
        xs = [interp]
        W1s = [Wts[0]]
    f = _mlp_max(xs, W1s, bs[0], Wts[1:], bs[1:], ns=1, ts=2048)
    return jnp.transpose(f, (0, 2, 1))


def kernel(pcd_xyz, rgb_xyz, rgb_features, params):
    xyz = pcd_xyz[..., :3]
    sa1_xyz, sa1_f, sa1_inds = _sa_module(xyz, None, 4096, 0.1, 64, params['sa1'])
    sa1_f_pm = jnp.transpose(sa1_f, (0, 2, 1))
    rgbf_pm = jnp.transpose(rgb_features, (0, 2, 1))
    feat2_pm = jnp.concatenate([
        jnp.concatenate([sa1_f_pm, jnp.zeros_like(sa1_f_pm)], axis=-1),
        jnp.concatenate([jnp.zeros_like(rgbf_pm), rgbf_pm], axis=-1)], axis=1)
    xyz2 = jnp.concatenate([sa1_xyz, rgb_xyz], axis=1)
    sa2_xyz, sa2_f, sa2_inds = _sa_module(xyz2, feat2_pm, 2048, 0.2, 64,
                                          params['sa2'])
    f = _fp_module(sa1_xyz, sa2_xyz, sa1_f, sa2_f, params['fp3'])
    f = _fp_module(pcd_xyz[..., :3], sa1_xyz, None, f, params['fp4'])
    return (sa1_inds, sa1_xyz, sa1_f, sa2_inds, sa2_xyz, sa2_f, f, pcd_xyz)


# speedup vs baseline: 12.7070x; 2.9995x over previous
import functools

import jax
import jax.numpy as jnp
from jax import lax
from jax.experimental import pallas as pl
from jax.experimental.pallas import tpu as pltpu, tpu_sc as plsc

_NC, _NS, _L = 2, 16, 16
_NW = _NC * _NS


def _mlp_body(*refs, nx, n_layers, ns, ts):
    out_ref = refs[-1]
    xs = refs[:nx]
    ws = refs[nx:-1]
    y = None
    for i in range(nx):
        t = jnp.dot(xs[i][0], ws[i][...], preferred_element_type=jnp.float32,
                    precision=lax.Precision.HIGHEST)
        y = t if y is None else y + t
    y = jnp.maximum(y + ws[nx][...], 0.0)
    k = nx + 1
    for _ in range(n_layers - 1):
        y = jnp.dot(y, ws[k][...], preferred_element_type=jnp.float32,
                    precision=lax.Precision.HIGHEST)
        y = jnp.maximum(y + ws[k + 1][...], 0.0)
        k += 2
    if ns > 1:
        y = y.reshape(ts, ns, y.shape[-1]).max(axis=1)
    out_ref[0] = y


def _mlp_max(xs, W1s, b1, Wr, br, ns, ts):
    nx = len(xs)
    b_, p, _ = xs[0].shape
    s = p // ns
    n_layers = 1 + len(Wr)
    cout = (Wr[-1] if Wr else W1s[0]).shape[1]
    grid = (b_, s // ts)
    in_specs = []
    args = []
    for x in xs:
        in_specs.append(pl.BlockSpec((1, ts * ns, x.shape[2]),
                                     lambda bi, pi: (bi, pi, 0)))
        args.append(x)
    weights = list(W1s) + [b1.reshape(1, -1)]
    for w, bias in zip(Wr, br):
        weights.append(w)
        weights.append(bias.reshape(1, -1))
    for w in weights:
        in_specs.append(pl.BlockSpec(w.shape, lambda bi, pi: (0, 0)))
        args.append(w)
    fn = pl.pallas_call(
        functools.partial(_mlp_body, nx=nx, n_layers=n_layers, ns=ns, ts=ts),
        grid=grid,
        in_specs=in_specs,
        out_specs=pl.BlockSpec((1, ts, cout), lambda bi, pi: (bi, pi, 0)),
        out_shape=jax.ShapeDtypeStruct((b_, s, cout), jnp.float32),
    )
    return fn(*args)


def _fps_body(xyz_ref, out_ref, cen_ref, *, npoint, n8):
    x = xyz_ref[0, 0]
    y = xyz_ref[0, 1]
    z = xyz_ref[0, 2]
    n = 8 * n8
    fidx = (lax.broadcasted_iota(jnp.int32, (8, n8), 0) * n8
            + lax.broadcasted_iota(jnp.int32, (8, n8), 1))

    def body(i, carry):
        dists, far = carry
        out_ref[0, 0, i] = far
        mask = fidx == far
        cx = jnp.sum(jnp.where(mask, x, 0.0))
        cy = jnp.sum(jnp.where(mask, y, 0.0))
        cz = jnp.sum(jnp.where(mask, z, 0.0))
        cen_ref[0, 0, i] = cx
        cen_ref[0, 1, i] = cy
        cen_ref[0, 2, i] = cz
        dx = x - cx
        dy = y - cy
        dz = z - cz
        d = (dx * dx + dy * dy) + dz * dz
        dists = jnp.minimum(dists, d)
        m = jnp.max(dists)
        far2 = jnp.min(jnp.where(dists == m, fidx, n))
        return dists, far2

    lax.fori_loop(0, npoint, body,
                  (jnp.full((8, n8), 1e10, jnp.float32), jnp.int32(0)),
                  unroll=2)


def _fps(xyz, npoint):
    b, n, _ = xyz.shape
    n8 = n // 8
    xr = jnp.transpose(xyz, (0, 2, 1)).reshape(b, 3, 8, n8)
    fn = pl.pallas_call(
        functools.partial(_fps_body, npoint=npoint, n8=n8),
        grid=(b,),
        in_specs=[pl.BlockSpec((1, 3, 8, n8), lambda bi: (bi, 0, 0, 0))],
        out_specs=[pl.BlockSpec((1, 1, npoint), lambda bi: (bi, 0, 0),
                                memory_space=pltpu.SMEM),
                   pl.BlockSpec((1, 3, npoint), lambda bi: (bi, 0, 0),
                                memory_space=pltpu.SMEM)],
        out_shape=[jax.ShapeDtypeStruct((b, 1, npoint), jnp.int32),
                   jax.ShapeDtypeStruct((b, 3, npoint), jnp.float32)],
    )
    inds, cen3 = fn(xr)
    return inds[:, 0, :], cen3


def _ball_query_sc(xyz3, cen3, radius, ns):
    B, _, n = xyz3.shape
    S = cen3.shape[2]
    gpt = (S // _L) // _NW
    r2 = jnp.float32(radius * radius)
    mesh = plsc.VectorSubcoreMesh(core_axis_name="c", subcore_axis_name="s")

    @functools.partial(
        pl.kernel,
        out_type=[jax.ShapeDtypeStruct((B * S * ns,), jnp.int32),
                  jax.ShapeDtypeStruct((B * S * ns,), jnp.float32),
                  jax.ShapeDtypeStruct((B * S * ns,), jnp.float32),
                  jax.ShapeDtypeStruct((B * S * ns,), jnp.float32)],
        mesh=mesh,
        scratch_types=[pltpu.VMEM((3, n), jnp.float32),
                       pltpu.VMEM((3, S), jnp.float32),
                       pltpu.VMEM((_L * ns,), jnp.int32),
                       pltpu.VMEM((_L * ns,), jnp.float32),
                       pltpu.VMEM((_L * ns,), jnp.float32),
                       pltpu.VMEM((_L * ns,), jnp.float32)],
        compiler_params=pltpu.CompilerParams(needs_layout_passes=False),
    )
    def bq(xyz_hbm, cen_hbm, idx_out, gxx_out, gxy_out, gxz_out,
           xyz_v, cen_v, oi, ogx, ogy, ogz):
        wid = lax.axis_index("s") * _NC + lax.axis_index("c")
        lanes = lax.iota(jnp.int32, _L)
        lbase = lanes * ns
        for b in range(B):
            pltpu.sync_copy(xyz_hbm.at[b], xyz_v)
            pltpu.sync_copy(cen_hbm.at[b], cen_v)

            def tbody(t, _):
                s0 = (t * _NW + wid) * _L
                cx = cen_v[0, pl.ds(s0, _L)]
                cy = cen_v[1, pl.ds(s0, _L)]
                cz = cen_v[2, pl.ds(s0, _L)]

                def jbody(jc, cnt):
                    j0 = jc * _L
                    xv = xyz_v[0, pl.ds(j0, _L)]
                    yv = xyz_v[1, pl.ds(j0, _L)]
                    zv = xyz_v[2, pl.ds(j0, _L)]
                    for l in range(_L):
                        dx = xv[l] - cx
                        dy = yv[l] - cy
                        dz = zv[l] - cz
                        d2 = (dx * dx + dy * dy) + dz * dz
                        hit = (d2 <= r2) & (cnt < ns)
                        pos = lbase + jnp.minimum(cnt, ns - 1)
                        jv = jnp.full((_L,), j0 + l, jnp.int32)
                        plsc.store_scatter(oi, [pos], jv, mask=hit)
                        plsc.store_scatter(ogx, [pos], dx, mask=hit)
                        plsc.store_scatter(ogy, [pos], dy, mask=hit)
                        plsc.store_scatter(ogz, [pos], dz, mask=hit)
                        cnt = cnt + hit.astype(jnp.int32)
                    return cnt

                cnt = lax.fori_loop(0, n // _L, jbody,
                                    jnp.zeros((_L,), jnp.int32))

                p0 = xyz_v[0, pl.ds(0, _L)]
                p1 = xyz_v[1, pl.ds(0, _L)]
                p2 = xyz_v[2, pl.ds(0, _L)]
                has = cnt > 0
                fi = jnp.where(has, plsc.load_gather(oi, [lbase]), 0)
                fx = jnp.where(has, plsc.load_gather(ogx, [lbase]), p0[0] - cx)
                fy = jnp.where(has, plsc.load_gather(ogy, [lbase]), p1[0] - cy)
                fz = jnp.where(has, plsc.load_gather(ogz, [lbase]), p2[0] - cz)

                def pbody(p, _):
                    m = cnt <= p
                    pv = lbase + p
                    plsc.store_scatter(oi, [pv], fi, mask=m)
                    plsc.store_scatter(ogx, [pv], fx, mask=m)
                    plsc.store_scatter(ogy, [pv], fy, mask=m)
                    plsc.store_scatter(ogz, [pv], fz, mask=m)
                    return 0

                lax.fori_loop(0, ns, pbody, 0)

                off = (b * S + s0) * ns
                pltpu.sync_copy(oi, idx_out.at[pl.ds(off, _L * ns)])
                pltpu.sync_copy(ogx, gxx_out.at[pl.ds(off, _L * ns)])
                pltpu.sync_copy(ogy, gxy_out.at[pl.ds(off, _L * ns)])
                pltpu.sync_copy(ogz, gxz_out.at[pl.ds(off, _L * ns)])
                return 0

            lax.fori_loop(0, gpt, tbody, 0)

    return bq(xyz3, cen3)


def _gather_rows_sc(table, idxg):
    V, C = table.shape
    R = idxg.shape[0]
    rpt = R // _NW
    nch = rpt // 128
    mesh = plsc.VectorSubcoreMesh(core_axis_name="c", subcore_axis_name="s")

    @functools.partial(
        pl.kernel,
        out_type=jax.ShapeDtypeStruct((R, C), jnp.float32),
        mesh=mesh,
        scratch_types=[pltpu.VMEM((128,), jnp.int32),
                       pltpu.VMEM((128, C), jnp.float32),
                       pltpu.SemaphoreType.DMA],
        compiler_params=pltpu.CompilerParams(needs_layout_passes=False),
    )
    def gk(table_hbm, idx_hbm, out_hbm, idx_v, rows_v, sem):
        wid = lax.axis_index("s") * _NC + lax.axis_index("c")
        base = wid * rpt

        def cbody(c, _):
            r0 = base + c * 128
            pltpu.sync_copy(idx_hbm.at[pl.ds(r0, 128)], idx_v)
            pltpu.async_copy(table_hbm.at[idx_v], rows_v, sem).wait()
            pltpu.sync_copy(rows_v, out_hbm.at[pl.ds(r0, 128), :])
            return 0

        lax.fori_loop(0, nch, cbody, 0)

    return gk(table, idxg)


def _sa_module(xyz, feats_pm, npoint, radius, nsample, p):
    Ws, bs = p
    b, n, _ = xyz.shape
    inds, cen3 = _fps(lax.stop_gradient(xyz), npoint)
    xyz3 = jnp.transpose(xyz, (0, 2, 1))
    idx, gxx, gxy, gxz = _ball_query_sc(xyz3, cen3, radius, nsample)
    idx = idx.reshape(b, npoint, nsample)
    gxf = jnp.stack([gxx, gxy, gxz], axis=-1).reshape(b, npoint * nsample, 3)
    Wts = [w.T for w in Ws]
    W1 = Wts[0]
    if feats_pm is not None:
        cf = feats_pm.shape[2]
        idxg = (idx.reshape(b, npoint * nsample)
                + (jnp.arange(b, dtype=jnp.int32) * n)[:, None]).reshape(-1)
        gf = _gather_rows_sc(feats_pm.reshape(b * n, cf), idxg)
        gf = gf.reshape(b, npoint * nsample, cf)
        xs = [gxf, gf]
        W1s = [W1[:3] / radius, W1[3:]]
    else:
        xs = [gxf]
        W1s = [W1 / radius]
    f = _mlp_max(xs, W1s, bs[0], Wts[1:], bs[1:], ns=nsample, ts=128)
    new_xyz = jnp.transpose(cen3, (0, 2, 1))
    return new_xyz, jnp.transpose(f, (0, 2, 1)), inds.astype(jnp.int64)


def _fp_body(*refs, nx, n_layers, tn, m):
    out_ref = refs[-1]
    u_ref, kT_ref, kfT_ref = refs[0], refs[1], refs[2]
    ws = refs[3 + (nx - 1):-1]
    u = u_ref[0]
    ux, uy, uz = u[:, 0:1], u[:, 1:2], u[:, 2:3]
    kx = kT_ref[0, 0:1, :]
    ky = kT_ref[0, 1:2, :]
    kz = kT_ref[0, 2:3, :]
    dx = ux - kx
    dy = uy - ky
    dz = uz - kz
    d2 = (dx * dx + dy * dy) + dz * dz
    colidx = lax.broadcasted_iota(jnp.int32, (tn, m), 1)
    big = jnp.float32(jnp.inf)
    mvs, iks = [], []
    for _ in range(3):
        mv = jnp.min(d2, axis=1, keepdims=True)
        ik = jnp.min(jnp.where(d2 == mv, colidx, m), axis=1, keepdims=True)
        mvs.append(mv)
        iks.append(ik)
        d2 = jnp.where(colidx == ik, big, d2)
    wk = [1.0 / (mv + 1e-8) for mv in mvs]
    wsum = (wk[0] + wk[1]) + wk[2]
    P = ((wk[0] / wsum) * (colidx == iks[0])
         + (wk[1] / wsum) * (colidx == iks[1])
         + (wk[2] / wsum) * (colidx == iks[2]))
    y = jnp.dot(P, kfT_ref[0], preferred_element_type=jnp.float32,
                precision=lax.Precision.HIGHEST)
    y = jnp.dot(y, ws[0][...], preferred_element_type=jnp.float32,
                precision=lax.Precision.HIGHEST)
    if nx == 2:
        y = y + jnp.dot(refs[3][0], ws[1][...],
                        preferred_element_type=jnp.float32,
                        precision=lax.Precision.HIGHEST)
    y = jnp.maximum(y + ws[nx][...], 0.0)
    k = nx + 1
    for _ in range(n_layers - 1):
        y = jnp.dot(y, ws[k][...], preferred_element_type=jnp.float32,
                    precision=lax.Precision.HIGHEST)
        y = jnp.maximum(y + ws[k + 1][...], 0.0)
        k += 2
    out_ref[0] = y


def _fp_module(unknown, known, uf_pm, kf_pm, p, tn=512):
    Ws, bs = p
    b, n, _ = unknown.shape
    m = known.shape[1]
    kT = jnp.transpose(known, (0, 2, 1))
    Wts = [w.T for w in Ws]
    ci = kf_pm.shape[2]
    nx = 2 if uf_pm is not None else 1
    W1s = [Wts[0][:ci]] + ([Wts[0][ci:]] if nx == 2 else [])
    n_layers = len(Wts)
    cout = Wts[-1].shape[1]
    in_specs = [pl.BlockSpec((1, tn, 3), lambda bi, pi: (bi, pi, 0)),
                pl.BlockSpec((1, 3, m), lambda bi, pi: (bi, 0, 0)),
                pl.BlockSpec((1, m, ci), lambda bi, pi: (bi, 0, 0))]
    args = [unknown, kT, kf_pm]
    if nx == 2:
        in_specs.append(pl.BlockSpec((1, tn, uf_pm.shape[2]),
                                     lambda bi, pi: (bi, pi, 0)))
        args.append(uf_pm)
    weights = list(W1s) + [bs[0].reshape(1, -1)]
    for w, bias in zip(Wts[1:], bs[1:]):
        weights.append(w)
        weights.append(bias.reshape(1, -1))
    for w in weights:
        in_specs.append(pl.BlockSpec(w.shape, lambda bi, pi: (0, 0)))
        args.append(w)
    fn = pl.pallas_call(
        functools.partial(_fp_body, nx=nx, n_layers=n_layers, tn=tn, m=m),
        grid=(b, n // tn),
        in_specs=in_specs,
        out_specs=pl.BlockSpec((1, tn, cout), lambda bi, pi: (bi, pi, 0)),
        out_shape=jax.ShapeDtypeStruct((b, n, cout), jnp.float32),
    )
    return fn(*args)


def kernel(pcd_xyz, rgb_xyz, rgb_features, params):
    xyz = pcd_xyz[..., :3]
    sa1_xyz, sa1_f, sa1_inds = _sa_module(xyz, None, 4096, 0.1, 64, params['sa1'])
    sa1_f_pm = jnp.transpose(sa1_f, (0, 2, 1))
    rgbf_pm = jnp.transpose(rgb_features, (0, 2, 1))
    feat2_pm = jnp.concatenate([
        jnp.concatenate([sa1_f_pm, jnp.zeros_like(sa1_f_pm)], axis=-1),
        jnp.concatenate([jnp.zeros_like(rgbf_pm), rgbf_pm], axis=-1)], axis=1)
    xyz2 = jnp.concatenate([sa1_xyz, rgb_xyz], axis=1)
    sa2_xyz, sa2_f, sa2_inds = _sa_module(xyz2, feat2_pm, 2048, 0.2, 64,
                                          params['sa2'])
    sa2_f_pm = jnp.transpose(sa2_f, (0, 2, 1))
    f_pm = _fp_module(sa1_xyz, sa2_xyz, sa1_f_pm, sa2_f_pm, params['fp3'])
    f_pm = _fp_module(pcd_xyz[..., :3], sa1_xyz, None, f_pm, params['fp4'])
    f = jnp.transpose(f_pm, (0, 2, 1))
    return (sa1_inds, sa1_xyz, sa1_f, sa2_inds, sa2_xyz, sa2_f, f, pcd_xyz)
